# Initial kernel scaffold; baseline (speedup 1.0000x reference)
import jax
import jax.numpy as jnp
from jax.experimental import pallas as pl


def kernel(x, A_idx, A_val, W1, b1, W2, b2):
    raise NotImplementedError("write your pallas kernel here")


# speedup vs baseline: 1.7830x; 1.7830x over previous
import functools

import jax
import jax.numpy as jnp
from jax import lax
from jax.experimental import pallas as pl
from jax.experimental.pallas import tpu as pltpu
from jax.experimental.pallas import tpu_sc as plsc

NC = 2
NS = 16
NW = NC * NS
K = 128


def _sc_spmm(xf, colf, rowp, valp, *, n_pad, e_pad, nb, c):
    ept = e_pad // NW
    steps = ept // K
    rows_per_tile = n_pad // NS

    mesh = plsc.VectorSubcoreMesh(core_axis_name="c", subcore_axis_name="s")

    @functools.partial(
        pl.kernel,
        out_type=jax.ShapeDtypeStruct((NC, nb, n_pad, c), jnp.float32),
        mesh=mesh,
        compiler_params=pltpu.CompilerParams(needs_layout_passes=False),
        scratch_types=[
            pltpu.VMEM((K,), jnp.int32),
            pltpu.VMEM((K,), jnp.int32),
            pltpu.VMEM((K,), jnp.float32),
            pltpu.VMEM((K, c), jnp.float32),
            pltpu.VMEM((K, c), jnp.float32),
            pltpu.VMEM_SHARED((n_pad, c), jnp.float32),
            pltpu.SemaphoreType.DMA,
        ],
    )
    def spmm(xf_h, colf_h, rowp_h, valp_h, out_h, colv, rowv, valv, rows_v,
             zbuf, acc, sem):
        cid = lax.axis_index("c")
        sid = lax.axis_index("s")
        tid = cid * NS + sid

        def zrow(i, carry):
            for j in range(c // 16):
                zbuf[i, pl.ds(j * 16, 16)] = jnp.zeros((16,), jnp.float32)
            return carry
        lax.fori_loop(0, K, zrow, 0)

        for b in range(nb):
            for z in range(rows_per_tile // K):
                pltpu.sync_copy(zbuf, acc.at[pl.ds(sid * rows_per_tile + z * K, K)])
            plsc.subcore_barrier()

            def step(st, carry):
                base = tid * ept + st * K
                pltpu.sync_copy(colf_h.at[pl.ds(b * e_pad + base, K)], colv)
                pltpu.sync_copy(rowp_h.at[pl.ds(base, K)], rowv)
                pltpu.sync_copy(valp_h.at[pl.ds(base, K)], valv)
                pltpu.async_copy(xf_h.at[colv], rows_v, sem).wait()

                def mul(e, carry2):
                    vs = plsc.load_gather(valv, [jnp.full((16,), e, jnp.int32)])
                    for j in range(c // 16):
                        rows_v[e, pl.ds(j * 16, 16)] = (
                            rows_v[e, pl.ds(j * 16, 16)] * vs)
                    return carry2
                lax.fori_loop(0, K, mul, 0)

                pltpu.sync_copy(rows_v, acc.at[rowv], add=True)
                return carry
            lax.fori_loop(0, steps, step, 0)
            plsc.subcore_barrier()

            for z in range(rows_per_tile // K):
                r0 = sid * rows_per_tile + z * K
                pltpu.sync_copy(acc.at[pl.ds(r0, K)], rows_v)
                pltpu.sync_copy(rows_v, out_h.at[cid, b, pl.ds(r0, K)])

    return spmm(xf, colf, rowp, valp)


def _mlp_kernel(yp_ref, w1t_ref, b1_ref, w2t_ref, b2_ref, out_ref):
    y = yp_ref[0, 0] + yp_ref[1, 0]
    h = jnp.maximum(
        jnp.dot(y, w1t_ref[...], preferred_element_type=jnp.float32)
        + b1_ref[...], 0.0)
    out_ref[0] = (
        jnp.dot(h, w2t_ref[...], preferred_element_type=jnp.float32)
        + b2_ref[...])


def _mlp(ypart, w1t, b1r, w2t, b2r, *, nb, n_pad, c, c_out, blk=512):
    grid = (nb, n_pad // blk)
    return pl.pallas_call(
        _mlp_kernel,
        grid=grid,
        in_specs=[
            pl.BlockSpec((2, 1, blk, c), lambda b, n: (0, b, n, 0)),
            pl.BlockSpec((c, c), lambda b, n: (0, 0)),
            pl.BlockSpec((1, c), lambda b, n: (0, 0)),
            pl.BlockSpec((c, c_out), lambda b, n: (0, 0)),
            pl.BlockSpec((1, c_out), lambda b, n: (0, 0)),
        ],
        out_specs=pl.BlockSpec((1, blk, c_out), lambda b, n: (b, n, 0)),
        out_shape=jax.ShapeDtypeStruct((nb, n_pad, c_out), jnp.float32),
    )(ypart, w1t, b1r, w2t, b2r)


def kernel(x, A_idx, A_val, W1, b1, W2, b2):
    nb, n, c = x.shape
    e = A_val.shape[0]
    c_out = W2.shape[0]

    n_pad = ((n + NS * K - 1) // (NS * K)) * (NS * K)
    e_pad = ((e + NW * K - 1) // (NW * K)) * (NW * K)

    row = A_idx[0].astype(jnp.int32)
    col = A_idx[1].astype(jnp.int32)
    pad = e_pad - e
    rowp = jnp.pad(row, (0, pad))
    valp = jnp.pad(A_val, (0, pad))
    colp = jnp.pad(col, (0, pad))
    colf = (colp[None, :] + (jnp.arange(nb, dtype=jnp.int32) * n)[:, None]
            ).reshape(nb * e_pad)
    xf = x.reshape(nb * n, c)

    ypart = _sc_spmm(xf, colf, rowp, valp, n_pad=n_pad, e_pad=e_pad, nb=nb, c=c)

    out = _mlp(ypart, W1.T, b1.reshape(1, -1), W2.T, b2.reshape(1, -1),
               nb=nb, n_pad=n_pad, c=c, c_out=c_out)
    return out[:, :n, :]
